# baseline (speedup 1.0000x reference)
import functools

import jax
import jax.numpy as jnp
from jax import lax
from jax.experimental import pallas as pl
from jax.experimental.pallas import tpu as pltpu
from jax.experimental.pallas import tpu_sc as plsc

TAU = 0.25
ALPHA = 3.0
GAMMA = 1.0
BALANCE = 0.7

N = 1048576
K = int(N * TAU)
NC = 2
NS = 16
NW = NC * NS
CHUNK = N // NW
NV = CHUNK // 16

_mesh = plsc.VectorSubcoreMesh(core_axis_name="c", subcore_axis_name="s")
_I32MIN = -2147483648


def _wid():
    return lax.axis_index("s") * NC + lax.axis_index("c")


def _ukey(pv):
    u = lax.bitcast_convert_type(pv, jnp.int32)
    s = lax.shift_right_arithmetic(u, 31)
    ik = (u ^ lax.shift_right_logical(s, 1)) - s
    return ik ^ jnp.int32(_I32MIN)


def _vany_pos(x):
    lane = lax.iota(jnp.int32, 16)
    for sh in (1, 2, 4, 8):
        g = x.at[lane ^ sh].get(mode="promise_in_bounds")
        x = jnp.maximum(x, g)
    return x[0] > 0


def _make_select(mode, interpret=False):
    first = mode != "buf"
    out_types = (
        jax.ShapeDtypeStruct((NW, 256), jnp.float32),
        jax.ShapeDtypeStruct((NW, CHUNK), jnp.int32),
        jax.ShapeDtypeStruct((NW, 16), jnp.int32),
    )
    if mode == "all":
        out_types = out_types[0]
    scratch = [
        pltpu.VMEM((CHUNK,), jnp.float32 if first else jnp.int32),
        pltpu.VMEM((128,), jnp.int32),
        pltpu.VMEM((256,), jnp.float32),
        pltpu.VMEM((16,), jnp.int32),
    ]
    if mode != "all":
        scratch.insert(1, pltpu.VMEM((CHUNK,), jnp.int32))

    @functools.partial(
        pl.kernel,
        out_type=out_types,
        mesh=_mesh,
        scratch_types=scratch,
        interpret=interpret,
    )
    def select_kernel(*refs):
        if mode == "all":
            (src_hbm, par_hbm, cnt_hbm, src_v, par_v, acc_v, sz_v) = refs
            dst_hbm = nsz_hbm = sizes_hbm = dst_v = None
        elif mode == "first":
            (src_hbm, par_hbm, cnt_hbm, dst_hbm, nsz_hbm,
             src_v, dst_v, par_v, acc_v, sz_v) = refs
            sizes_hbm = None
        else:
            (src_hbm, sizes_hbm, par_hbm, cnt_hbm, dst_hbm, nsz_hbm,
             src_v, dst_v, par_v, acc_v, sz_v) = refs
        w = _wid()
        pltpu.sync_copy(par_hbm, par_v)
        prefix_v = par_v[pl.ds(0, 16)]
        s4_v = par_v[pl.ds(16, 16)]
        sh_v = par_v[pl.ds(32, 16)]
        dead_v = par_v[pl.ds(48, 16)]
        tl_v = par_v[pl.ds(64, 16)]
        th_v = par_v[pl.ds(80, 16)]
        if first:
            pltpu.sync_copy(src_hbm.at[pl.ds(w * CHUNK, CHUNK)], src_v)
            nch = NV // 8
        else:
            pltpu.sync_copy(sizes_hbm.at[w], sz_v)
            nvec = sz_v[...][0]
            nch = lax.shift_right_logical(nvec + 7, 3)
            pltpu.sync_copy(src_hbm.at[w], src_v)
        z = jnp.zeros((16,), jnp.float32)
        zi = jnp.zeros((16,), jnp.int32)

        def body(ci, carry):
            accs, off = carry
            accl = zi
            acch = zi
            if mode != "all":
                offe = pl.multiple_of(off * 16, 16)
            for u in range(8):
                iv = ci * 8 + u
                src = src_v[pl.ds(pl.multiple_of(iv * 16, 16), 16)]
                uk = _ukey(src) if first else src
                d = jnp.bitwise_and(lax.shift_right_logical(uk, sh_v), 15)
                if mode == "all":
                    dd = d
                else:
                    match = lax.shift_right_logical(uk, s4_v) == prefix_v
                    dd = jnp.where(match, d, -1)
                    if mode == "buf":
                        valid_s = jnp.where(iv < nvec, 15, -1)
                        dd = jnp.minimum(dd, valid_s)
                    dst_v[pl.ds(offe + 16 * u, 16)] = jnp.where(
                        dd >= 0, uk, dead_v
                    )
                j = jnp.minimum(dd, 7) + 1
                h = jnp.maximum(dd - 7, 0)
                accl = accl + tl_v.at[j].get(mode="promise_in_bounds")
                acch = acch + th_v.at[h].get(mode="promise_in_bounds")
            accs = tuple(
                acc
                + jnp.bitwise_and(
                    lax.shift_right_logical(accl if b < 8 else acch,
                                            4 * (b % 8)),
                    15,
                ).astype(jnp.float32)
                for b, acc in enumerate(accs)
            )
            if mode == "all":
                return (accs, off)
            off = off + jnp.where(_vany_pos(accl), 8, 0).astype(jnp.int32)
            return (accs, off)

        (accs, off) = lax.fori_loop(0, nch, body, ((z,) * 16, jnp.int32(0)))
        for b in range(16):
            acc_v[pl.ds(16 * b, 16)] = accs[b]
        pltpu.sync_copy(acc_v, cnt_hbm.at[w])
        if mode != "all":
            sz_v[pl.ds(0, 16)] = jnp.zeros((16,), jnp.int32) + off
            pltpu.sync_copy(sz_v, nsz_hbm.at[w])
            for s in range(16):
                @pl.when(s * 128 < off)
                def _():
                    pltpu.sync_copy(
                        dst_v.at[pl.ds(s * 2048, 2048)],
                        dst_hbm.at[w, pl.ds(s * 2048, 2048)],
                    )

    return select_kernel


def _make_sums(interpret=False):
    @functools.partial(
        pl.kernel,
        out_type=jax.ShapeDtypeStruct((NW, 64), jnp.float32),
        mesh=_mesh,
        scratch_types=[
            pltpu.VMEM((CHUNK,), jnp.float32),
            pltpu.VMEM((CHUNK,), jnp.float32),
            pltpu.VMEM((16,), jnp.float32),
            pltpu.VMEM((64,), jnp.float32),
        ],
        interpret=interpret,
    )
    def sums_kernel(pred_hbm, tgt_hbm, par_hbm, out_hbm, dp, dt, par_v, acc_v):
        w = _wid()
        pltpu.sync_copy(pred_hbm.at[pl.ds(w * CHUNK, CHUNK)], dp)
        pltpu.sync_copy(tgt_hbm.at[pl.ds(w * CHUNK, CHUNK)], dt)
        pltpu.sync_copy(par_hbm, par_v)
        tau_v = par_v[...]
        z = jnp.zeros((16,), jnp.float32)

        def body(i, carry):
            sa, sg, st, ct = carry
            base = pl.multiple_of(i * 128, 128)
            for u in range(8):
                off = base + 16 * u
                pv = dp[pl.ds(off, 16)]
                tv = dt[pl.ds(off, 16)]
                d = pv - tv
                mse = d * d
                gt = pv > tau_v
                eq = pv == tau_v
                sa = sa + mse
                sg = sg + jnp.where(gt, mse, 0.0)
                st = st + jnp.where(eq, mse, 0.0)
                ct = ct + jnp.where(eq, 1.0, 0.0)
            return (sa, sg, st, ct)

        sa, sg, st, ct = lax.fori_loop(0, NV // 8, body, (z, z, z, z))
        acc_v[pl.ds(0, 16)] = sa
        acc_v[pl.ds(16, 16)] = sg
        acc_v[pl.ds(32, 16)] = st
        acc_v[pl.ds(48, 16)] = ct
        pltpu.sync_copy(acc_v, out_hbm.at[w])

    return sums_kernel


def _vprefix(x):
    lane = lax.iota(jnp.int32, 16)
    for sh in (1, 2, 4, 8):
        g = x.at[jnp.maximum(lane - sh, 0)].get(mode="promise_in_bounds")
        x = x + jnp.where(lane >= sh, g, 0.0)
    return x


def _make_fix(interpret=False):
    @functools.partial(
        pl.kernel,
        out_type=jax.ShapeDtypeStruct((16,), jnp.float32),
        mesh=_mesh,
        scratch_types=[
            pltpu.VMEM((CHUNK,), jnp.float32),
            pltpu.VMEM((CHUNK,), jnp.float32),
            pltpu.VMEM((32,), jnp.float32),
            pltpu.VMEM((16,), jnp.int32),
            pltpu.VMEM((16,), jnp.float32),
        ],
        interpret=interpret,
    )
    def fix_kernel(pred_hbm, tgt_hbm, parf_hbm, pari_hbm, out_hbm, dp, dt, parf_v, pari_v, acc_v):
        w = _wid()
        pltpu.sync_copy(pari_hbm, pari_v)
        fw = pari_v[...][0]

        @pl.when(w == fw)
        def _():
            pltpu.sync_copy(pred_hbm.at[pl.ds(w * CHUNK, CHUNK)], dp)
            pltpu.sync_copy(tgt_hbm.at[pl.ds(w * CHUNK, CHUNK)], dt)
            pltpu.sync_copy(parf_hbm, parf_v)
            tau_v = parf_v[pl.ds(0, 16)]
            mf_v = parf_v[pl.ds(16, 16)]
            acc_v[pl.ds(0, 16)] = jnp.zeros((16,), jnp.float32)

            def body(i, cntv):
                off = pl.multiple_of(i * 16, 16)
                pv = dp[pl.ds(off, 16)]
                tv = dt[pl.ds(off, 16)]
                eq = pv == tau_v
                eqf = jnp.where(eq, 1.0, 0.0)
                cs = _vprefix(eqf) + cntv
                take = jnp.logical_and(cs <= mf_v, eq)
                d = pv - tv
                plsc.addupdate(acc_v.at[pl.ds(0, 16)], jnp.where(take, d * d, 0.0))
                return jnp.zeros((16,), jnp.float32) + cs[15]

            lax.fori_loop(0, NV, body, jnp.zeros((16,), jnp.float32))
            pltpu.sync_copy(acc_v, out_hbm)

    return fix_kernel


_sel0 = _make_select("all")
_sel1 = _make_select("first")
_selN = _make_select("buf")
_sums = _make_sums()
_fix = _make_fix()

_TL = [sum(1 << (4 * b) for b in range(j)) for j in range(9)] + [0] * 7

def _bcast16(x, dtype=jnp.int32):
    return jnp.full((16,), 1, dtype) * x.astype(dtype)


def kernel(pred, target):
    def pick(counts, kk, prefix):
        c = counts.reshape(NW, 16, 16).sum(axis=(0, 2)).astype(jnp.int32)
        b = jnp.max(
            jnp.where(c >= kk, jnp.arange(16, dtype=jnp.int32), -1)
        ).astype(jnp.int32)
        c_next = jnp.concatenate([c[1:], jnp.zeros((1,), jnp.int32)])
        return kk - c_next[b], jnp.left_shift(prefix, 4) | b

    prefix = jnp.zeros((), jnp.int32)
    kk = jnp.full((), K, jnp.int32)
    tl = jnp.asarray(_TL, jnp.int32)
    tables = jnp.concatenate([tl, tl])

    def mkpar(prefix, shift, dead):
        return jnp.concatenate(
            [
                _bcast16(prefix),
                jnp.full((16,), shift + 4, jnp.int32),
                jnp.full((16,), shift, jnp.int32),
                _bcast16(dead),
                tables,
                jnp.zeros((32,), jnp.int32),
            ]
        )

    par0 = mkpar(jnp.zeros((), jnp.int32), 28, jnp.zeros((), jnp.int32))
    kk, prefix = pick(_sel0(pred, par0), kk, prefix)

    srcbuf = sizes = None
    for p in range(1, 8):
        shift = 28 - 4 * p
        dead = jnp.left_shift(prefix ^ 1, shift + 4)
        par = mkpar(prefix, shift, dead)
        if p == 1:
            counts, srcbuf, sizes = _sel1(pred, par)
        else:
            counts, srcbuf, sizes = _selN(srcbuf, sizes, par)
        kk, prefix = pick(counts, kk, prefix)

    j = kk
    ik = prefix ^ jnp.int32(_I32MIN)
    u = jnp.where(ik < 0, (ik - 1) ^ jnp.int32(0x7FFFFFFF), ik)
    tau = lax.bitcast_convert_type(u, jnp.float32)
    tau_v = _bcast16(tau, jnp.float32)

    part = _sums(pred, target, tau_v).reshape(NW, 4, 16)
    s_all = part[:, 0, :].sum()
    s_gt = part[:, 1, :].sum()
    st_w = part[:, 2, :].sum(axis=1)
    ct_w = part[:, 3, :].sum(axis=1).astype(jnp.int32)

    offs = jnp.concatenate([jnp.zeros((1,), jnp.int32), jnp.cumsum(ct_w)[:-1]])
    m_w = jnp.clip(j - offs, 0, ct_w)
    tie_full = jnp.where(m_w == ct_w, st_w, 0.0).sum()
    partial = jnp.logical_and(m_w > 0, m_w < ct_w)
    need = jnp.any(partial)
    fw = jnp.argmax(partial).astype(jnp.int32)
    mf = jnp.max(jnp.where(partial, m_w, 0)).astype(jnp.int32)

    def with_fix():
        pari = jnp.zeros((16,), jnp.int32).at[0].set(fw)
        parf = jnp.concatenate(
            [tau_v, jnp.full((16,), 1.0, jnp.float32) * mf.astype(jnp.float32)]
        )
        return jnp.sum(_fix(pred, target, parf, pari))

    fixsum = lax.cond(need, with_fix, lambda: jnp.float32(0.0))

    top = s_gt + tie_full + fixsum
    top_loss = ALPHA * top / K
    overall = (ALPHA * top + GAMMA * (s_all - top)) / N
    return BALANCE * top_loss + (1.0 - BALANCE) * overall


# speedup vs baseline: 1.0066x; 1.0066x over previous
import functools

import jax
import jax.numpy as jnp
from jax import lax
from jax.experimental import pallas as pl
from jax.experimental.pallas import tpu as pltpu
from jax.experimental.pallas import tpu_sc as plsc

TAU = 0.25
ALPHA = 3.0
GAMMA = 1.0
BALANCE = 0.7

N = 1048576
K = int(N * TAU)
NC = 2
NS = 16
NW = NC * NS
CHUNK = N // NW
NV = CHUNK // 16

_mesh = plsc.VectorSubcoreMesh(core_axis_name="c", subcore_axis_name="s")
_I32MIN = -2147483648


def _wid():
    return lax.axis_index("s") * NC + lax.axis_index("c")


def _ukey(pv):
    u = lax.bitcast_convert_type(pv, jnp.int32)
    s = lax.shift_right_arithmetic(u, 31)
    ik = (u ^ lax.shift_right_logical(s, 1)) - s
    return ik ^ jnp.int32(_I32MIN)


def _vany_pos(x):
    lane = lax.iota(jnp.int32, 16)
    for sh in (1, 2, 4, 8):
        g = x.at[lane ^ sh].get(mode="promise_in_bounds")
        x = jnp.maximum(x, g)
    return x[0] > 0


def _make_select(mode, interpret=False):
    first = mode != "buf"
    out_types = (
        jax.ShapeDtypeStruct((NW, 256), jnp.float32),
        jax.ShapeDtypeStruct((NW, CHUNK), jnp.int32),
        jax.ShapeDtypeStruct((NW, 16), jnp.int32),
    )
    if mode == "all":
        out_types = out_types[0]
    scratch = [
        pltpu.VMEM((CHUNK,), jnp.float32 if first else jnp.int32),
        pltpu.VMEM((128,), jnp.int32),
        pltpu.VMEM((256,), jnp.float32),
        pltpu.VMEM((16,), jnp.int32),
    ]
    if mode != "all":
        scratch.insert(1, pltpu.VMEM((CHUNK,), jnp.int32))

    @functools.partial(
        pl.kernel,
        out_type=out_types,
        mesh=_mesh,
        scratch_types=scratch,
        interpret=interpret,
    )
    def select_kernel(*refs):
        if mode == "all":
            (src_hbm, par_hbm, cnt_hbm, src_v, par_v, acc_v, sz_v) = refs
            dst_hbm = nsz_hbm = sizes_hbm = dst_v = None
        elif mode == "first":
            (src_hbm, par_hbm, cnt_hbm, dst_hbm, nsz_hbm,
             src_v, dst_v, par_v, acc_v, sz_v) = refs
            sizes_hbm = None
        else:
            (src_hbm, sizes_hbm, par_hbm, cnt_hbm, dst_hbm, nsz_hbm,
             src_v, dst_v, par_v, acc_v, sz_v) = refs
        w = _wid()
        pltpu.sync_copy(par_hbm, par_v)
        prefix_v = par_v[pl.ds(0, 16)]
        s4_v = par_v[pl.ds(16, 16)]
        sh_v = par_v[pl.ds(32, 16)]
        dead_v = par_v[pl.ds(48, 16)]
        tl_v = par_v[pl.ds(64, 16)]
        th_v = par_v[pl.ds(80, 16)]
        if first:
            pltpu.sync_copy(src_hbm.at[pl.ds(w * CHUNK, CHUNK)], src_v)
            nch = NV // 8
        else:
            pltpu.sync_copy(sizes_hbm.at[w], sz_v)
            nvec = sz_v[...][0]
            nch = lax.shift_right_logical(nvec + 7, 3)
            pltpu.sync_copy(src_hbm.at[w], src_v)
        z = jnp.zeros((16,), jnp.float32)
        zi = jnp.zeros((16,), jnp.int32)

        def body(ci, carry):
            accs, off = carry
            accl = zi
            acch = zi
            if mode != "all":
                offe = pl.multiple_of(off * 16, 16)
            for u in range(8):
                iv = ci * 8 + u
                src = src_v[pl.ds(pl.multiple_of(iv * 16, 16), 16)]
                uk = _ukey(src) if first else src
                d = jnp.bitwise_and(lax.shift_right_logical(uk, sh_v), 15)
                if mode == "all":
                    dd = d
                else:
                    match = lax.shift_right_logical(uk, s4_v) == prefix_v
                    dd = jnp.where(match, d, -1)
                    if mode == "buf":
                        valid_s = jnp.where(iv < nvec, 15, -1)
                        dd = jnp.minimum(dd, valid_s)
                    dst_v[pl.ds(offe + 16 * u, 16)] = jnp.where(
                        dd >= 0, uk, dead_v
                    )
                j = jnp.minimum(dd, 7) + 1
                h = jnp.maximum(dd - 7, 0)
                accl = accl + tl_v.at[j].get(mode="promise_in_bounds")
                acch = acch + th_v.at[h].get(mode="promise_in_bounds")
            accs = tuple(
                acc
                + jnp.bitwise_and(
                    lax.shift_right_logical(accl if b < 8 else acch,
                                            4 * (b % 8)),
                    15,
                ).astype(jnp.float32)
                for b, acc in enumerate(accs)
            )
            if mode == "all":
                return (accs, off)
            off = off + jnp.where(_vany_pos(accl), 8, 0).astype(jnp.int32)
            return (accs, off)

        (accs, off) = lax.fori_loop(0, nch, body, ((z,) * 16, jnp.int32(0)))
        for b in range(16):
            acc_v[pl.ds(16 * b, 16)] = accs[b]
        pltpu.sync_copy(acc_v, cnt_hbm.at[w])
        if mode != "all":
            sz_v[pl.ds(0, 16)] = jnp.zeros((16,), jnp.int32) + off
            pltpu.sync_copy(sz_v, nsz_hbm.at[w])
            for s in range(16):
                @pl.when(s * 128 < off)
                def _():
                    pltpu.sync_copy(
                        dst_v.at[pl.ds(s * 2048, 2048)],
                        dst_hbm.at[w, pl.ds(s * 2048, 2048)],
                    )

    return select_kernel


def _make_sums(interpret=False):
    @functools.partial(
        pl.kernel,
        out_type=jax.ShapeDtypeStruct((NW, 64), jnp.float32),
        mesh=_mesh,
        scratch_types=[
            pltpu.VMEM((CHUNK,), jnp.float32),
            pltpu.VMEM((CHUNK,), jnp.float32),
            pltpu.VMEM((16,), jnp.float32),
            pltpu.VMEM((64,), jnp.float32),
            pltpu.SemaphoreType.DMA,
            pltpu.SemaphoreType.DMA,
        ],
        interpret=interpret,
    )
    def sums_kernel(pred_hbm, tgt_hbm, par_hbm, out_hbm, dp, dt, par_v, acc_v,
                    sem1, sem2):
        w = _wid()
        c1 = pltpu.async_copy(pred_hbm.at[pl.ds(w * CHUNK, CHUNK)], dp, sem1)
        c2 = pltpu.async_copy(tgt_hbm.at[pl.ds(w * CHUNK, CHUNK)], dt, sem2)
        pltpu.sync_copy(par_hbm, par_v)
        c1.wait()
        c2.wait()
        tau_v = par_v[...]
        z = jnp.zeros((16,), jnp.float32)

        def body(i, carry):
            sa, sg, st, ct = carry
            base = pl.multiple_of(i * 128, 128)
            for u in range(8):
                off = base + 16 * u
                pv = dp[pl.ds(off, 16)]
                tv = dt[pl.ds(off, 16)]
                d = pv - tv
                mse = d * d
                gt = pv > tau_v
                eq = pv == tau_v
                sa = sa + mse
                sg = sg + jnp.where(gt, mse, 0.0)
                st = st + jnp.where(eq, mse, 0.0)
                ct = ct + jnp.where(eq, 1.0, 0.0)
            return (sa, sg, st, ct)

        sa, sg, st, ct = lax.fori_loop(0, NV // 8, body, (z, z, z, z))
        acc_v[pl.ds(0, 16)] = sa
        acc_v[pl.ds(16, 16)] = sg
        acc_v[pl.ds(32, 16)] = st
        acc_v[pl.ds(48, 16)] = ct
        pltpu.sync_copy(acc_v, out_hbm.at[w])

    return sums_kernel


def _vprefix(x):
    lane = lax.iota(jnp.int32, 16)
    for sh in (1, 2, 4, 8):
        g = x.at[jnp.maximum(lane - sh, 0)].get(mode="promise_in_bounds")
        x = x + jnp.where(lane >= sh, g, 0.0)
    return x


def _make_fix(interpret=False):
    @functools.partial(
        pl.kernel,
        out_type=jax.ShapeDtypeStruct((16,), jnp.float32),
        mesh=_mesh,
        scratch_types=[
            pltpu.VMEM((CHUNK,), jnp.float32),
            pltpu.VMEM((CHUNK,), jnp.float32),
            pltpu.VMEM((32,), jnp.float32),
            pltpu.VMEM((16,), jnp.int32),
            pltpu.VMEM((16,), jnp.float32),
        ],
        interpret=interpret,
    )
    def fix_kernel(pred_hbm, tgt_hbm, parf_hbm, pari_hbm, out_hbm, dp, dt, parf_v, pari_v, acc_v):
        w = _wid()
        pltpu.sync_copy(pari_hbm, pari_v)
        fw = pari_v[...][0]

        @pl.when(w == fw)
        def _():
            pltpu.sync_copy(pred_hbm.at[pl.ds(w * CHUNK, CHUNK)], dp)
            pltpu.sync_copy(tgt_hbm.at[pl.ds(w * CHUNK, CHUNK)], dt)
            pltpu.sync_copy(parf_hbm, parf_v)
            tau_v = parf_v[pl.ds(0, 16)]
            mf_v = parf_v[pl.ds(16, 16)]
            acc_v[pl.ds(0, 16)] = jnp.zeros((16,), jnp.float32)

            def body(i, cntv):
                off = pl.multiple_of(i * 16, 16)
                pv = dp[pl.ds(off, 16)]
                tv = dt[pl.ds(off, 16)]
                eq = pv == tau_v
                eqf = jnp.where(eq, 1.0, 0.0)
                cs = _vprefix(eqf) + cntv
                take = jnp.logical_and(cs <= mf_v, eq)
                d = pv - tv
                plsc.addupdate(acc_v.at[pl.ds(0, 16)], jnp.where(take, d * d, 0.0))
                return jnp.zeros((16,), jnp.float32) + cs[15]

            lax.fori_loop(0, NV, body, jnp.zeros((16,), jnp.float32))
            pltpu.sync_copy(acc_v, out_hbm)

    return fix_kernel


_sel0 = _make_select("all")
_sel1 = _make_select("first")
_selN = _make_select("buf")
_sums = _make_sums()
_fix = _make_fix()

_TL = [sum(1 << (4 * b) for b in range(j)) for j in range(9)] + [0] * 7

def _bcast16(x, dtype=jnp.int32):
    return jnp.full((16,), 1, dtype) * x.astype(dtype)


def kernel(pred, target):
    def pick(counts, kk, prefix):
        c = counts.reshape(NW, 16, 16).sum(axis=(0, 2)).astype(jnp.int32)
        b = jnp.max(
            jnp.where(c >= kk, jnp.arange(16, dtype=jnp.int32), -1)
        ).astype(jnp.int32)
        c_next = jnp.concatenate([c[1:], jnp.zeros((1,), jnp.int32)])
        return kk - c_next[b], jnp.left_shift(prefix, 4) | b

    prefix = jnp.zeros((), jnp.int32)
    kk = jnp.full((), K, jnp.int32)
    tl = jnp.asarray(_TL, jnp.int32)
    tables = jnp.concatenate([tl, tl])

    def mkpar(prefix, shift, dead):
        return jnp.concatenate(
            [
                _bcast16(prefix),
                jnp.full((16,), shift + 4, jnp.int32),
                jnp.full((16,), shift, jnp.int32),
                _bcast16(dead),
                tables,
                jnp.zeros((32,), jnp.int32),
            ]
        )

    par0 = mkpar(jnp.zeros((), jnp.int32), 28, jnp.zeros((), jnp.int32))
    kk, prefix = pick(_sel0(pred, par0), kk, prefix)

    srcbuf = sizes = None
    for p in range(1, 8):
        shift = 28 - 4 * p
        dead = jnp.left_shift(prefix ^ 1, shift + 4)
        par = mkpar(prefix, shift, dead)
        if p == 1:
            counts, srcbuf, sizes = _sel1(pred, par)
        else:
            counts, srcbuf, sizes = _selN(srcbuf, sizes, par)
        kk, prefix = pick(counts, kk, prefix)

    j = kk
    ik = prefix ^ jnp.int32(_I32MIN)
    u = jnp.where(ik < 0, (ik - 1) ^ jnp.int32(0x7FFFFFFF), ik)
    tau = lax.bitcast_convert_type(u, jnp.float32)
    tau_v = _bcast16(tau, jnp.float32)

    part = _sums(pred, target, tau_v).reshape(NW, 4, 16)
    s_all = part[:, 0, :].sum()
    s_gt = part[:, 1, :].sum()
    st_w = part[:, 2, :].sum(axis=1)
    ct_w = part[:, 3, :].sum(axis=1).astype(jnp.int32)

    offs = jnp.concatenate([jnp.zeros((1,), jnp.int32), jnp.cumsum(ct_w)[:-1]])
    m_w = jnp.clip(j - offs, 0, ct_w)
    tie_full = jnp.where(m_w == ct_w, st_w, 0.0).sum()
    partial = jnp.logical_and(m_w > 0, m_w < ct_w)
    need = jnp.any(partial)
    fw = jnp.argmax(partial).astype(jnp.int32)
    mf = jnp.max(jnp.where(partial, m_w, 0)).astype(jnp.int32)

    def with_fix():
        pari = jnp.zeros((16,), jnp.int32).at[0].set(fw)
        parf = jnp.concatenate(
            [tau_v, jnp.full((16,), 1.0, jnp.float32) * mf.astype(jnp.float32)]
        )
        return jnp.sum(_fix(pred, target, parf, pari))

    fixsum = lax.cond(need, with_fix, lambda: jnp.float32(0.0))

    top = s_gt + tie_full + fixsum
    top_loss = ALPHA * top / K
    overall = (ALPHA * top + GAMMA * (s_all - top)) / N
    return BALANCE * top_loss + (1.0 - BALANCE) * overall


# speedup vs baseline: 1.0069x; 1.0003x over previous
import functools

import jax
import jax.numpy as jnp
from jax import lax
from jax.experimental import pallas as pl
from jax.experimental.pallas import tpu as pltpu
from jax.experimental.pallas import tpu_sc as plsc

TAU = 0.25
ALPHA = 3.0
GAMMA = 1.0
BALANCE = 0.7

N = 1048576
K = int(N * TAU)
NC = 2
NS = 16
NW = NC * NS
CHUNK = N // NW
NV = CHUNK // 16

_mesh = plsc.VectorSubcoreMesh(core_axis_name="c", subcore_axis_name="s")
_I32MIN = -2147483648


def _wid():
    return lax.axis_index("s") * NC + lax.axis_index("c")


def _ukey(pv):
    u = lax.bitcast_convert_type(pv, jnp.int32)
    s = lax.shift_right_arithmetic(u, 31)
    ik = (u ^ lax.shift_right_logical(s, 1)) - s
    return ik ^ jnp.int32(_I32MIN)


def _vany_pos(x):
    lane = lax.iota(jnp.int32, 16)
    for sh in (1, 2, 4, 8):
        g = x.at[lane ^ sh].get(mode="promise_in_bounds")
        x = jnp.maximum(x, g)
    return x[0] > 0


def _make_select(mode):
    first = mode != "buf"
    out_types = (
        jax.ShapeDtypeStruct((NW, 256), jnp.float32),
        jax.ShapeDtypeStruct((NW, CHUNK), jnp.int32),
        jax.ShapeDtypeStruct((NW, 16), jnp.int32),
    )
    if mode == "all":
        out_types = out_types[0]
    scratch = [
        pltpu.VMEM((CHUNK,), jnp.float32 if first else jnp.int32),
        pltpu.VMEM((128,), jnp.int32),
        pltpu.VMEM((256,), jnp.float32),
        pltpu.VMEM((16,), jnp.int32),
    ]
    if mode != "all":
        scratch.insert(1, pltpu.VMEM((CHUNK,), jnp.int32))

    @functools.partial(
        pl.kernel,
        out_type=out_types,
        mesh=_mesh,
        scratch_types=scratch,
    )
    def select_kernel(*refs):
        if mode == "all":
            (src_hbm, par_hbm, cnt_hbm, src_v, par_v, acc_v, sz_v) = refs
            dst_hbm = nsz_hbm = sizes_hbm = dst_v = None
        elif mode == "first":
            (src_hbm, par_hbm, cnt_hbm, dst_hbm, nsz_hbm,
             src_v, dst_v, par_v, acc_v, sz_v) = refs
            sizes_hbm = None
        else:
            (src_hbm, sizes_hbm, par_hbm, cnt_hbm, dst_hbm, nsz_hbm,
             src_v, dst_v, par_v, acc_v, sz_v) = refs
        w = _wid()
        pltpu.sync_copy(par_hbm, par_v)
        prefix_v = par_v[pl.ds(0, 16)]
        s4_v = par_v[pl.ds(16, 16)]
        sh_v = par_v[pl.ds(32, 16)]
        dead_v = par_v[pl.ds(48, 16)]
        tl_v = par_v[pl.ds(64, 16)]
        th_v = par_v[pl.ds(80, 16)]
        if first:
            pltpu.sync_copy(src_hbm.at[pl.ds(w * CHUNK, CHUNK)], src_v)
            nch = NV // 8
        else:
            pltpu.sync_copy(sizes_hbm.at[w], sz_v)
            nvec = sz_v[...][0]
            nch = lax.shift_right_logical(nvec + 7, 3)
            pltpu.sync_copy(src_hbm.at[w], src_v)
        z = jnp.zeros((16,), jnp.float32)
        zi = jnp.zeros((16,), jnp.int32)

        def body(ci, carry):
            accs, off = carry
            accl = zi
            acch = zi
            if mode != "all":
                offe = pl.multiple_of(off * 16, 16)
            for u in range(8):
                iv = ci * 8 + u
                src = src_v[pl.ds(pl.multiple_of(iv * 16, 16), 16)]
                uk = _ukey(src) if first else src
                d = jnp.bitwise_and(lax.shift_right_logical(uk, sh_v), 15)
                if mode == "all":
                    dd = d
                else:
                    match = lax.shift_right_logical(uk, s4_v) == prefix_v
                    dd = jnp.where(match, d, -1)
                    if mode == "buf":
                        valid_s = jnp.where(iv < nvec, 15, -1)
                        dd = jnp.minimum(dd, valid_s)
                    dst_v[pl.ds(offe + 16 * u, 16)] = jnp.where(
                        dd >= 0, uk, dead_v
                    )
                j = jnp.minimum(dd, 7) + 1
                h = jnp.maximum(dd - 7, 0)
                accl = accl + tl_v.at[j].get(mode="promise_in_bounds")
                acch = acch + th_v.at[h].get(mode="promise_in_bounds")
            accs = tuple(
                acc
                + jnp.bitwise_and(
                    lax.shift_right_logical(accl if b < 8 else acch,
                                            4 * (b % 8)),
                    15,
                ).astype(jnp.float32)
                for b, acc in enumerate(accs)
            )
            if mode == "all":
                return (accs, off)
            off = off + jnp.where(_vany_pos(accl), 8, 0).astype(jnp.int32)
            return (accs, off)

        (accs, off) = lax.fori_loop(0, nch, body, ((z,) * 16, jnp.int32(0)))
        for b in range(16):
            acc_v[pl.ds(16 * b, 16)] = accs[b]
        pltpu.sync_copy(acc_v, cnt_hbm.at[w])
        if mode != "all":
            sz_v[pl.ds(0, 16)] = jnp.zeros((16,), jnp.int32) + off
            pltpu.sync_copy(sz_v, nsz_hbm.at[w])
            for s in range(16):
                @pl.when(s * 128 < off)
                def _():
                    pltpu.sync_copy(
                        dst_v.at[pl.ds(s * 2048, 2048)],
                        dst_hbm.at[w, pl.ds(s * 2048, 2048)],
                    )

    return select_kernel


def _make_sums():
    @functools.partial(
        pl.kernel,
        out_type=jax.ShapeDtypeStruct((NW, 64), jnp.float32),
        mesh=_mesh,
        scratch_types=[
            pltpu.VMEM((CHUNK,), jnp.float32),
            pltpu.VMEM((CHUNK,), jnp.float32),
            pltpu.VMEM((16,), jnp.float32),
            pltpu.VMEM((64,), jnp.float32),
            pltpu.SemaphoreType.DMA,
            pltpu.SemaphoreType.DMA,
        ],
    )
    def sums_kernel(pred_hbm, tgt_hbm, par_hbm, out_hbm, dp, dt, par_v, acc_v,
                    sem1, sem2):
        w = _wid()
        c1 = pltpu.async_copy(pred_hbm.at[pl.ds(w * CHUNK, CHUNK)], dp, sem1)
        c2 = pltpu.async_copy(tgt_hbm.at[pl.ds(w * CHUNK, CHUNK)], dt, sem2)
        pltpu.sync_copy(par_hbm, par_v)
        c1.wait()
        c2.wait()
        tau_v = par_v[...]
        z = jnp.zeros((16,), jnp.float32)

        def body(i, carry):
            sa, sg, st, ct = carry
            base = pl.multiple_of(i * 128, 128)
            for u in range(8):
                off = base + 16 * u
                pv = dp[pl.ds(off, 16)]
                tv = dt[pl.ds(off, 16)]
                d = pv - tv
                mse = d * d
                gt = pv > tau_v
                eq = pv == tau_v
                sa = sa + mse
                sg = sg + jnp.where(gt, mse, 0.0)
                st = st + jnp.where(eq, mse, 0.0)
                ct = ct + jnp.where(eq, 1.0, 0.0)
            return (sa, sg, st, ct)

        sa, sg, st, ct = lax.fori_loop(0, NV // 8, body, (z, z, z, z))
        acc_v[pl.ds(0, 16)] = sa
        acc_v[pl.ds(16, 16)] = sg
        acc_v[pl.ds(32, 16)] = st
        acc_v[pl.ds(48, 16)] = ct
        pltpu.sync_copy(acc_v, out_hbm.at[w])

    return sums_kernel


def _vprefix(x):
    lane = lax.iota(jnp.int32, 16)
    for sh in (1, 2, 4, 8):
        g = x.at[jnp.maximum(lane - sh, 0)].get(mode="promise_in_bounds")
        x = x + jnp.where(lane >= sh, g, 0.0)
    return x


def _make_fix():
    @functools.partial(
        pl.kernel,
        out_type=jax.ShapeDtypeStruct((16,), jnp.float32),
        mesh=_mesh,
        scratch_types=[
            pltpu.VMEM((CHUNK,), jnp.float32),
            pltpu.VMEM((CHUNK,), jnp.float32),
            pltpu.VMEM((32,), jnp.float32),
            pltpu.VMEM((16,), jnp.int32),
            pltpu.VMEM((16,), jnp.float32),
        ],
    )
    def fix_kernel(pred_hbm, tgt_hbm, parf_hbm, pari_hbm, out_hbm, dp, dt, parf_v, pari_v, acc_v):
        w = _wid()
        pltpu.sync_copy(pari_hbm, pari_v)
        fw = pari_v[...][0]

        @pl.when(w == fw)
        def _():
            pltpu.sync_copy(pred_hbm.at[pl.ds(w * CHUNK, CHUNK)], dp)
            pltpu.sync_copy(tgt_hbm.at[pl.ds(w * CHUNK, CHUNK)], dt)
            pltpu.sync_copy(parf_hbm, parf_v)
            tau_v = parf_v[pl.ds(0, 16)]
            mf_v = parf_v[pl.ds(16, 16)]
            acc_v[pl.ds(0, 16)] = jnp.zeros((16,), jnp.float32)

            def body(i, cntv):
                off = pl.multiple_of(i * 16, 16)
                pv = dp[pl.ds(off, 16)]
                tv = dt[pl.ds(off, 16)]
                eq = pv == tau_v
                eqf = jnp.where(eq, 1.0, 0.0)
                cs = _vprefix(eqf) + cntv
                take = jnp.logical_and(cs <= mf_v, eq)
                d = pv - tv
                plsc.addupdate(acc_v.at[pl.ds(0, 16)], jnp.where(take, d * d, 0.0))
                return jnp.zeros((16,), jnp.float32) + cs[15]

            lax.fori_loop(0, NV, body, jnp.zeros((16,), jnp.float32))
            pltpu.sync_copy(acc_v, out_hbm)

    return fix_kernel


_sel0 = _make_select("all")
_sel1 = _make_select("first")
_selN = _make_select("buf")
_sums = _make_sums()
_fix = _make_fix()

_TL = [sum(1 << (4 * b) for b in range(j)) for j in range(9)] + [0] * 7

def _bcast16(x, dtype=jnp.int32):
    return jnp.full((16,), 1, dtype) * x.astype(dtype)


def kernel(pred, target):
    def pick(counts, kk, prefix):
        c = counts.reshape(NW, 16, 16).sum(axis=(0, 2)).astype(jnp.int32)
        b = jnp.max(
            jnp.where(c >= kk, jnp.arange(16, dtype=jnp.int32), -1)
        ).astype(jnp.int32)
        c_next = jnp.concatenate([c[1:], jnp.zeros((1,), jnp.int32)])
        return kk - c_next[b], jnp.left_shift(prefix, 4) | b

    prefix = jnp.zeros((), jnp.int32)
    kk = jnp.full((), K, jnp.int32)
    tl = jnp.asarray(_TL, jnp.int32)
    tables = jnp.concatenate([tl, tl])

    def mkpar(prefix, shift, dead):
        return jnp.concatenate(
            [
                _bcast16(prefix),
                jnp.full((16,), shift + 4, jnp.int32),
                jnp.full((16,), shift, jnp.int32),
                _bcast16(dead),
                tables,
                jnp.zeros((32,), jnp.int32),
            ]
        )

    par0 = mkpar(jnp.zeros((), jnp.int32), 28, jnp.zeros((), jnp.int32))
    kk, prefix = pick(_sel0(pred, par0), kk, prefix)

    srcbuf = sizes = None
    for p in range(1, 8):
        shift = 28 - 4 * p
        dead = jnp.left_shift(prefix ^ 1, shift + 4)
        par = mkpar(prefix, shift, dead)
        if p == 1:
            counts, srcbuf, sizes = _sel1(pred, par)
        else:
            counts, srcbuf, sizes = _selN(srcbuf, sizes, par)
        kk, prefix = pick(counts, kk, prefix)

    j = kk
    ik = prefix ^ jnp.int32(_I32MIN)
    u = jnp.where(ik < 0, (ik - 1) ^ jnp.int32(0x7FFFFFFF), ik)
    tau = lax.bitcast_convert_type(u, jnp.float32)
    tau_v = _bcast16(tau, jnp.float32)

    part = _sums(pred, target, tau_v).reshape(NW, 4, 16)
    s_all = part[:, 0, :].sum()
    s_gt = part[:, 1, :].sum()
    st_w = part[:, 2, :].sum(axis=1)
    ct_w = part[:, 3, :].sum(axis=1).astype(jnp.int32)

    offs = jnp.concatenate([jnp.zeros((1,), jnp.int32), jnp.cumsum(ct_w)[:-1]])
    m_w = jnp.clip(j - offs, 0, ct_w)
    tie_full = jnp.where(m_w == ct_w, st_w, 0.0).sum()
    partial = jnp.logical_and(m_w > 0, m_w < ct_w)
    need = jnp.any(partial)
    fw = jnp.argmax(partial).astype(jnp.int32)
    mf = jnp.max(jnp.where(partial, m_w, 0)).astype(jnp.int32)

    def with_fix():
        pari = jnp.zeros((16,), jnp.int32).at[0].set(fw)
        parf = jnp.concatenate(
            [tau_v, jnp.full((16,), 1.0, jnp.float32) * mf.astype(jnp.float32)]
        )
        return jnp.sum(_fix(pred, target, parf, pari))

    fixsum = lax.cond(need, with_fix, lambda: jnp.float32(0.0))

    top = s_gt + tie_full + fixsum
    top_loss = ALPHA * top / K
    overall = (ALPHA * top + GAMMA * (s_all - top)) / N
    return BALANCE * top_loss + (1.0 - BALANCE) * overall
